# baseline (speedup 1.0000x reference)
import jax
import jax.numpy as jnp
from jax import lax
from jax.experimental import pallas as pl
from jax.experimental.pallas import tpu as pltpu
from jax.experimental.pallas import tpu_sc as plsc

_B = 16384
_C = 5
_D = 64
_NC = 2
_NS = 16
_NW = _NC * _NS
_BPW = _B // _NW
_CB = 128
_NCHUNK = _BPW // _CB
_PAIRS = _CB * _C
_W = 2 * _D
_V = 1000000
_NT = _V // _W
_TPW = _NT // _NW
_NTAIL = _V - _NT * _W


_SL = 256
_NT2 = _V // _SL
_TPW2 = _NT2 // _NW


def _transpose_blk(inblk, outblk, npairs):
    iota = lax.iota(jnp.int32, 16)

    @plsc.parallel_loop(0, npairs, unroll=16)
    def col_pair(ql):
        col0 = jnp.full((16,), 2 * ql, jnp.int32)
        col1 = col0 + 1
        for m in range(8):
            rows = 16 * (m % 4) + iota
            outblk[ql, pl.ds(16 * m, 16)] = plsc.load_gather(
                inblk, [rows, col0 if m < 4 else col1])


def _in_cp(src_view, T, inblk, sem):
    return pltpu.make_async_copy(
        src_view.at[:, pl.ds(pl.multiple_of(T * _SL, _W), _SL)], inblk, sem)


def _out_cp(dst128, T, outblk, sem):
    return pltpu.make_async_copy(
        outblk, dst128.at[pl.ds(pl.multiple_of(T * (_SL // 2), 8),
                                _SL // 2)], sem)


def _k1_body(ctabT_hbm, c128_hbm, in0, in1, out0, out1,
             sin0, sin1, sout0, sout1, sem):
    wid = lax.axis_index("s") * _NC + lax.axis_index("c")
    _in_cp(ctabT_hbm, wid, in0, sin0).start()

    def pair(j, carry):
        t0 = wid + _NW * 2 * j
        t1 = t0 + _NW
        tp = lax.min(t1 + _NW, _NT2 - 1)
        _in_cp(ctabT_hbm, t1, in1, sin1).start()
        _in_cp(ctabT_hbm, t0, in0, sin0).wait()

        @pl.when(j > 0)
        def _():
            _out_cp(c128_hbm, t0, out0, sout0).wait()

        _transpose_blk(in0, out0, _SL // 2)
        _out_cp(c128_hbm, t0, out0, sout0).start()

        _in_cp(ctabT_hbm, tp, in0, sin0).start()
        _in_cp(ctabT_hbm, t1, in1, sin1).wait()

        @pl.when(j > 0)
        def _():
            _out_cp(c128_hbm, t1, out1, sout1).wait()

        _transpose_blk(in1, out1, _SL // 2)
        _out_cp(c128_hbm, t1, out1, sout1).start()
        return carry

    lax.fori_loop(0, _TPW2 // 2, pair, 0)
    tlast = wid + _NW * (_TPW2 - 1)
    _out_cp(c128_hbm, tlast, out0, sout0).wait()
    _out_cp(c128_hbm, tlast, out1, sout1).wait()
    _in_cp(ctabT_hbm, lax.min(wid + _NW * _TPW2, _NT2 - 1), in0, sin0).wait()

    @pl.when(wid < _NT2 - _TPW2 * _NW)
    def _():
        T = _TPW2 * _NW + wid
        _in_cp(ctabT_hbm, T, in0, sin0).start()
        _in_cp(ctabT_hbm, T, in0, sin0).wait()
        _transpose_blk(in0, out0, _SL // 2)
        _out_cp(c128_hbm, T, out0, sout0).start()
        _out_cp(c128_hbm, T, out0, sout0).wait()

    @pl.when(wid == 4)
    def _():
        t = lax.min(wid, 0) + _NT
        cp_in = pltpu.make_async_copy(
            ctabT_hbm.at[:, pl.ds(pl.multiple_of(t * _W, _W), _W)],
            in0.at[:, pl.ds(0, _W)], sem)
        cp_in.start()
        cp_in.wait()
        _transpose_blk(in0, out0, _W // 2)
        cp_out = pltpu.make_async_copy(
            out0.at[pl.ds(0, _W // 2)],
            c128_hbm.at[pl.ds(pl.multiple_of(t * (_W // 2), 8), _W // 2)],
            sem)
        cp_out.start()
        cp_out.wait()


def _k2_body(tgt_hbm, ctx_hbm, bmap_hbm, ttab_hbm, c128_hbm, out_hbm,
             tidx, cidx, cidx_hi, bmapv, trows, crows, outbuf, sem, gsem):
    wid = lax.axis_index("s") * _NC + lax.axis_index("c")
    base = wid * _BPW
    pltpu.sync_copy(bmap_hbm, bmapv)
    iota = lax.iota(jnp.int32, 16)
    for chunk in range(_NCHUNK):
        b0 = base + chunk * _CB
        pltpu.sync_copy(tgt_hbm.at[pl.ds(b0, _CB)], tidx)
        pltpu.sync_copy(ctx_hbm.at[pl.ds(b0 * _C, _PAIRS)], cidx)

        def prep_c(m, carry):
            v = lax.shift_right_logical(cidx[pl.ds(m * 16, 16)], 1)
            cidx_hi[lax.shift_right_logical(m, 3),
                    pl.ds((m % 8) * 16, 16)] = v
            return carry

        lax.fori_loop(0, _PAIRS // 16, prep_c, 0)

        cps = []
        for j in range(_C):
            cp = pltpu.make_async_copy(c128_hbm.at[cidx_hi.at[j]],
                                       crows.at[pl.ds(j * _CB, _CB)], gsem)
            cp.start()
            cps.append(cp)

        def fire_t(g, carry):
            v = tidx[pl.ds(g * 16, 16)]
            for i in range(16):
                pltpu.make_async_copy(ttab_hbm.at[pl.ds(v[i], 1)],
                                      trows.at[pl.ds(g * 16 + i, 1)],
                                      sem).start()
            return carry

        lax.fori_loop(0, _CB // 16, fire_t, 0)

        def drain_t(k, carry):
            pltpu.make_async_copy(ttab_hbm.at[pl.ds(0, 1)],
                                  trows.at[pl.ds(k, 1)], sem).wait()
            return carry

        lax.fori_loop(0, _CB, drain_t, 0)
        for cp in cps:
            cp.wait()

        def body(g, carry):
            p0 = g * 16
            b_l = bmapv[pl.ds(p0, 16)]
            craw = cidx[pl.ds(p0, 16)]
            cbase = (p0 + iota) * _W + (craw & 1) * _D
            acc = jnp.zeros((16,), jnp.float32)
            for e in range(_D):
                ce = cbase + e
                wv = plsc.load_gather(trows, [b_l, jnp.full((16,), e,
                                                            jnp.int32)])
                cv = plsc.load_gather(crows, [lax.shift_right_logical(ce, 7),
                                              ce & 127])
                acc = acc + wv * cv
            outbuf[pl.ds(p0, 16)] = acc
            return carry

        lax.fori_loop(0, _PAIRS // 16, body, 0)
        pltpu.sync_copy(outbuf, out_hbm.at[pl.ds(b0 * _C, _PAIRS)])


@jax.jit
def kernel(target, context, target_table, context_table):
    tgt = target.astype(jnp.int32)
    ctx = context.reshape(-1).astype(jnp.int32)
    bmap = (jnp.arange(_PAIRS, dtype=jnp.int32) // _C)
    mesh = plsc.VectorSubcoreMesh(core_axis_name="c", subcore_axis_name="s",
                                  num_cores=_NC, num_subcores=_NS)
    params = pltpu.CompilerParams(needs_layout_passes=False,
                                  use_tc_tiling_on_sc=True)
    c128 = pl.kernel(
        _k1_body,
        out_type=jax.ShapeDtypeStruct((_V // 2 + 32, _W), jnp.float32),
        mesh=mesh,
        compiler_params=params,
        scratch_types=[
            pltpu.VMEM((_D, _SL), jnp.float32),
            pltpu.VMEM((_D, _SL), jnp.float32),
            pltpu.VMEM((_SL // 2, _W), jnp.float32),
            pltpu.VMEM((_SL // 2, _W), jnp.float32),
            pltpu.SemaphoreType.DMA,
            pltpu.SemaphoreType.DMA,
            pltpu.SemaphoreType.DMA,
            pltpu.SemaphoreType.DMA,
            pltpu.SemaphoreType.DMA,
        ],
    )(context_table.T)
    out_flat = pl.kernel(
        _k2_body,
        out_type=jax.ShapeDtypeStruct((_B * _C,), jnp.float32),
        mesh=mesh,
        compiler_params=params,
        scratch_types=[
            pltpu.VMEM((_CB,), jnp.int32),
            pltpu.VMEM((_PAIRS,), jnp.int32),
            pltpu.VMEM((_C, _CB), jnp.int32),
            pltpu.VMEM((_PAIRS,), jnp.int32),
            pltpu.VMEM((_CB, _D), jnp.float32),
            pltpu.VMEM((_PAIRS, _W), jnp.float32),
            pltpu.VMEM((_PAIRS,), jnp.float32),
            pltpu.SemaphoreType.DMA,
            pltpu.SemaphoreType.DMA,
        ],
    )(tgt, ctx, bmap, target_table, c128)
    return out_flat.reshape(_B, _C)


# speedup vs baseline: 1.1270x; 1.1270x over previous
import jax
import jax.numpy as jnp
from jax import lax
from jax.experimental import pallas as pl
from jax.experimental.pallas import tpu as pltpu
from jax.experimental.pallas import tpu_sc as plsc

_B = 16384
_C = 5
_D = 64
_NC = 2
_NS = 16
_NW = _NC * _NS
_BPW = _B // _NW
_CB = 128
_NCHUNK = _BPW // _CB
_PAIRS = _CB * _C
_W = 2 * _D
_V = 1000000
_NT = _V // _W
_TPW = _NT // _NW
_NTAIL = _V - _NT * _W


_SL = 256
_NT2 = _V // _SL
_TPW2 = _NT2 // _NW


def _transpose_blk(inblk, outblk, npairs):
    iota = lax.iota(jnp.int32, 16)

    @plsc.parallel_loop(0, npairs, unroll=16)
    def col_pair(ql):
        col0 = jnp.full((16,), 2 * ql, jnp.int32)
        col1 = col0 + 1
        for m in range(8):
            rows = 16 * (m % 4) + iota
            outblk[ql, pl.ds(16 * m, 16)] = plsc.load_gather(
                inblk, [rows, col0 if m < 4 else col1])


def _in_cp(src_view, T, inblk, sem):
    return pltpu.make_async_copy(
        src_view.at[:, pl.ds(pl.multiple_of(T * _SL, _W), _SL)],
        inblk.at[:, pl.ds(0, _SL)], sem)


def _out_cp(dst128, T, outblk, sem):
    return pltpu.make_async_copy(
        outblk, dst128.at[pl.ds(pl.multiple_of(T * (_SL // 2), 8),
                                _SL // 2)], sem)


def _k1_body(ctabT_hbm, c128_hbm, in0, in1, out0, out1,
             sin0, sin1, sout0, sout1, sem):
    wid = lax.axis_index("s") * _NC + lax.axis_index("c")
    _in_cp(ctabT_hbm, wid, in0, sin0).start()

    def pair(j, carry):
        t0 = wid + _NW * 2 * j
        t1 = t0 + _NW
        tp = lax.min(t1 + _NW, _NT2 - 1)
        _in_cp(ctabT_hbm, t1, in1, sin1).start()
        _in_cp(ctabT_hbm, t0, in0, sin0).wait()

        @pl.when(j > 0)
        def _():
            _out_cp(c128_hbm, t0, out0, sout0).wait()

        _transpose_blk(in0, out0, _SL // 2)
        _out_cp(c128_hbm, t0, out0, sout0).start()

        _in_cp(ctabT_hbm, tp, in0, sin0).start()
        _in_cp(ctabT_hbm, t1, in1, sin1).wait()

        @pl.when(j > 0)
        def _():
            _out_cp(c128_hbm, t1, out1, sout1).wait()

        _transpose_blk(in1, out1, _SL // 2)
        _out_cp(c128_hbm, t1, out1, sout1).start()
        return carry

    lax.fori_loop(0, _TPW2 // 2, pair, 0)
    tlast = wid + _NW * (_TPW2 - 1)
    _out_cp(c128_hbm, tlast, out0, sout0).wait()
    _out_cp(c128_hbm, tlast, out1, sout1).wait()
    _in_cp(ctabT_hbm, lax.min(wid + _NW * _TPW2, _NT2 - 1), in0, sin0).wait()

    @pl.when(wid < _NT2 - _TPW2 * _NW)
    def _():
        T = _TPW2 * _NW + wid
        _in_cp(ctabT_hbm, T, in0, sin0).start()
        _in_cp(ctabT_hbm, T, in0, sin0).wait()
        _transpose_blk(in0, out0, _SL // 2)
        _out_cp(c128_hbm, T, out0, sout0).start()
        _out_cp(c128_hbm, T, out0, sout0).wait()

    @pl.when(wid == 4)
    def _():
        t = lax.min(wid, 0) + _NT
        cp_in = pltpu.make_async_copy(
            ctabT_hbm.at[:, pl.ds(pl.multiple_of(t * _W, _W), _W)],
            in0.at[:, pl.ds(0, _W)], sem)
        cp_in.start()
        cp_in.wait()
        _transpose_blk(in0, out0, _W // 2)
        cp_out = pltpu.make_async_copy(
            out0.at[pl.ds(0, _W // 2)],
            c128_hbm.at[pl.ds(pl.multiple_of(t * (_W // 2), 8), _W // 2)],
            sem)
        cp_out.start()
        cp_out.wait()


def _k2_body(tgt_hbm, ctx_hbm, bmap_hbm, ttab_hbm, c128_hbm, out_hbm,
             tidx, cidx, cidx_hi, bmapv, trows, crows, outbuf, sem, gsem):
    wid = lax.axis_index("s") * _NC + lax.axis_index("c")
    base = wid * _BPW
    pltpu.sync_copy(bmap_hbm, bmapv)
    iota = lax.iota(jnp.int32, 16)
    for chunk in range(_NCHUNK):
        b0 = base + chunk * _CB
        pltpu.sync_copy(tgt_hbm.at[pl.ds(b0, _CB)], tidx)
        pltpu.sync_copy(ctx_hbm.at[pl.ds(b0 * _C, _PAIRS)], cidx)

        def prep_c(m, carry):
            v = lax.shift_right_logical(cidx[pl.ds(m * 16, 16)], 1)
            cidx_hi[lax.shift_right_logical(m, 3),
                    pl.ds((m % 8) * 16, 16)] = v
            return carry

        lax.fori_loop(0, _PAIRS // 16, prep_c, 0)

        cps = []
        for j in range(_C):
            cp = pltpu.make_async_copy(c128_hbm.at[cidx_hi.at[j]],
                                       crows.at[pl.ds(j * _CB, _CB)], gsem)
            cp.start()
            cps.append(cp)

        def fire_t(g, carry):
            v = tidx[pl.ds(g * 16, 16)]
            for i in range(16):
                pltpu.make_async_copy(ttab_hbm.at[pl.ds(v[i], 1)],
                                      trows.at[pl.ds(g * 16 + i, 1)],
                                      sem).start()
            return carry

        lax.fori_loop(0, _CB // 16, fire_t, 0)

        def drain_t(k, carry):
            pltpu.make_async_copy(ttab_hbm.at[pl.ds(0, 1)],
                                  trows.at[pl.ds(k, 1)], sem).wait()
            return carry

        lax.fori_loop(0, _CB, drain_t, 0)
        for cp in cps:
            cp.wait()

        def body(g, carry):
            p0 = g * 16
            b_l = bmapv[pl.ds(p0, 16)]
            craw = cidx[pl.ds(p0, 16)]
            cbase = (p0 + iota) * _W + (craw & 1) * _D
            acc = jnp.zeros((16,), jnp.float32)
            for e in range(_D):
                ev = (e + iota) & (_D - 1)
                ce = cbase + ev
                wv = plsc.load_gather(trows, [b_l, ev])
                cv = plsc.load_gather(crows, [lax.shift_right_logical(ce, 7),
                                              ce & 127])
                acc = acc + wv * cv
            outbuf[pl.ds(p0, 16)] = acc
            return carry

        lax.fori_loop(0, _PAIRS // 16, body, 0)
        pltpu.sync_copy(outbuf, out_hbm.at[pl.ds(b0 * _C, _PAIRS)])


@jax.jit
def kernel(target, context, target_table, context_table):
    tgt = target.astype(jnp.int32)
    ctx = context.reshape(-1).astype(jnp.int32)
    bmap = (jnp.arange(_PAIRS, dtype=jnp.int32) // _C)
    mesh = plsc.VectorSubcoreMesh(core_axis_name="c", subcore_axis_name="s",
                                  num_cores=_NC, num_subcores=_NS)
    params = pltpu.CompilerParams(needs_layout_passes=False,
                                  use_tc_tiling_on_sc=True)
    c128 = pl.kernel(
        _k1_body,
        out_type=jax.ShapeDtypeStruct((_V // 2 + 32, _W), jnp.float32),
        mesh=mesh,
        compiler_params=params,
        scratch_types=[
            pltpu.VMEM((_D, _SL + 1), jnp.float32),
            pltpu.VMEM((_D, _SL + 1), jnp.float32),
            pltpu.VMEM((_SL // 2, _W), jnp.float32),
            pltpu.VMEM((_SL // 2, _W), jnp.float32),
            pltpu.SemaphoreType.DMA,
            pltpu.SemaphoreType.DMA,
            pltpu.SemaphoreType.DMA,
            pltpu.SemaphoreType.DMA,
            pltpu.SemaphoreType.DMA,
        ],
    )(context_table.T)
    out_flat = pl.kernel(
        _k2_body,
        out_type=jax.ShapeDtypeStruct((_B * _C,), jnp.float32),
        mesh=mesh,
        compiler_params=params,
        scratch_types=[
            pltpu.VMEM((_CB,), jnp.int32),
            pltpu.VMEM((_PAIRS,), jnp.int32),
            pltpu.VMEM((_C, _CB), jnp.int32),
            pltpu.VMEM((_PAIRS,), jnp.int32),
            pltpu.VMEM((_CB, _D), jnp.float32),
            pltpu.VMEM((_PAIRS, _W), jnp.float32),
            pltpu.VMEM((_PAIRS,), jnp.float32),
            pltpu.SemaphoreType.DMA,
            pltpu.SemaphoreType.DMA,
        ],
    )(tgt, ctx, bmap, target_table, c128)
    return out_flat.reshape(_B, _C)


# speedup vs baseline: 1.4333x; 1.2718x over previous
import jax
import jax.numpy as jnp
from jax import lax
from jax.experimental import pallas as pl
from jax.experimental.pallas import tpu as pltpu
from jax.experimental.pallas import tpu_sc as plsc

_B = 16384
_C = 5
_D = 64
_NC = 2
_NS = 16
_NW = _NC * _NS
_BPW = _B // _NW
_CB = 128
_NCHUNK = _BPW // _CB
_PAIRS = _CB * _C


def _sc_body(tgt_hbm, ctx_hbm, ttab_hbm, ctab_hbm, out_hbm,
             tidx, cidx, trows, crows, outbuf, sem):
    wid = lax.axis_index("s") * _NC + lax.axis_index("c")
    base = wid * _BPW
    for chunk in range(_NCHUNK):
        b0 = base + chunk * _CB
        pltpu.sync_copy(tgt_hbm.at[pl.ds(b0, _CB)], tidx)
        pltpu.sync_copy(ctx_hbm.at[pl.ds(b0 * _C, _PAIRS)], cidx)

        def fire_t(g, carry):
            v = tidx[pl.ds(g * 16, 16)]
            for i in range(16):
                pltpu.make_async_copy(ttab_hbm.at[pl.ds(v[i], 1)],
                                      trows.at[pl.ds(g * 16 + i, 1)],
                                      sem).start()
            return carry

        lax.fori_loop(0, _CB // 16, fire_t, 0)

        def fire_c(g, carry):
            v = cidx[pl.ds(g * 16, 16)]
            for i in range(16):
                pltpu.make_async_copy(ctab_hbm.at[pl.ds(v[i], 1)],
                                      crows.at[pl.ds(g * 16 + i, 1)],
                                      sem).start()
            return carry

        lax.fori_loop(0, _PAIRS // 16, fire_c, 0)

        def drain_t(k, carry):
            pltpu.make_async_copy(ttab_hbm.at[pl.ds(0, 1)],
                                  trows.at[pl.ds(k, 1)], sem).wait()
            return carry

        lax.fori_loop(0, _CB, drain_t, 0)

        def drain_c(k, carry):
            pltpu.make_async_copy(ctab_hbm.at[pl.ds(0, 1)],
                                  crows.at[pl.ds(k, 1)], sem).wait()
            return carry

        lax.fori_loop(0, _PAIRS, drain_c, 0)

        def zero(i, carry):
            outbuf[pl.ds(i * 16, 16)] = jnp.zeros((16,), jnp.float32)
            return carry

        lax.fori_loop(0, _PAIRS // 16, zero, 0)

        def body(b, carry):
            w = [trows[b, pl.ds(16 * j, 16)] for j in range(_D // 16)]
            for c in range(_C):
                row = b * _C + c
                acc = w[0] * crows[row, pl.ds(0, 16)]
                for j in range(1, _D // 16):
                    acc = acc + w[j] * crows[row, pl.ds(16 * j, 16)]
                idx = jnp.full((16,), row, jnp.int32)
                plsc.addupdate_scatter(outbuf, [idx], acc)
            return carry

        lax.fori_loop(0, _CB, body, 0)
        pltpu.sync_copy(outbuf, out_hbm.at[pl.ds(b0 * _C, _PAIRS)])


@jax.jit
def kernel(target, context, target_table, context_table):
    tgt = target.astype(jnp.int32)
    ctx = context.reshape(-1).astype(jnp.int32)
    mesh = plsc.VectorSubcoreMesh(core_axis_name="c", subcore_axis_name="s",
                                  num_cores=_NC, num_subcores=_NS)
    out_flat = pl.kernel(
        _sc_body,
        out_type=jax.ShapeDtypeStruct((_B * _C,), jnp.float32),
        mesh=mesh,
        compiler_params=pltpu.CompilerParams(needs_layout_passes=False,
                                             use_tc_tiling_on_sc=True),
        scratch_types=[
            pltpu.VMEM((_CB,), jnp.int32),
            pltpu.VMEM((_PAIRS,), jnp.int32),
            pltpu.VMEM((_CB, _D), jnp.float32),
            pltpu.VMEM((_PAIRS, _D), jnp.float32),
            pltpu.VMEM((_PAIRS,), jnp.float32),
            pltpu.SemaphoreType.DMA,
        ],
    )(tgt, ctx, target_table, context_table)
    return out_flat.reshape(_B, _C)


# speedup vs baseline: 1.9143x; 1.3357x over previous
import jax
import jax.numpy as jnp
from jax import lax
from jax.experimental import pallas as pl
from jax.experimental.pallas import tpu as pltpu
from jax.experimental.pallas import tpu_sc as plsc

_B = 16384
_C = 5
_D = 64
_NC = 2
_NS = 16
_NW = _NC * _NS
_BPW = _B // _NW
_CB = 128
_NCHUNK = _BPW // _CB
_PAIRS = _CB * _C


def _sc_body(tgt_hbm, ctx_hbm, ttab_hbm, ctab3_hbm, out_hbm,
             tidx, cidx, trows, crows, outbuf, sem):
    ctab_hbm = ctab3_hbm.reshape(1000000, _D)
    wid = lax.axis_index("s") * _NC + lax.axis_index("c")
    base = wid * _BPW
    for chunk in range(_NCHUNK):
        b0 = base + chunk * _CB
        pltpu.sync_copy(tgt_hbm.at[pl.ds(b0, _CB)], tidx)
        pltpu.sync_copy(ctx_hbm.at[pl.ds(b0 * _C, _PAIRS)], cidx)

        def fire_t(g, carry):
            v = tidx[pl.ds(g * 16, 16)]
            for i in range(16):
                pltpu.make_async_copy(ttab_hbm.at[pl.ds(v[i], 1)],
                                      trows.at[pl.ds(g * 16 + i, 1)],
                                      sem).start()
            return carry

        lax.fori_loop(0, _CB // 16, fire_t, 0)

        def fire_c(g, carry):
            v = cidx[pl.ds(g * 16, 16)]
            for i in range(16):
                pltpu.make_async_copy(ctab_hbm.at[pl.ds(v[i], 1)],
                                      crows.at[pl.ds(g * 16 + i, 1)],
                                      sem).start()
            return carry

        lax.fori_loop(0, _PAIRS // 16, fire_c, 0)

        def drain_t(k, carry):
            pltpu.make_async_copy(ttab_hbm.at[pl.ds(0, 1)],
                                  trows.at[pl.ds(k, 1)], sem).wait()
            return carry

        lax.fori_loop(0, _CB, drain_t, 0)

        def drain_c(k, carry):
            pltpu.make_async_copy(ctab_hbm.at[pl.ds(0, 1)],
                                  crows.at[pl.ds(k, 1)], sem).wait()
            return carry

        lax.fori_loop(0, _PAIRS, drain_c, 0)

        def zero(i, carry):
            outbuf[pl.ds(i * 16, 16)] = jnp.zeros((16,), jnp.float32)
            return carry

        lax.fori_loop(0, _PAIRS // 16, zero, 0)

        def body(b, carry):
            w = [trows[b, pl.ds(16 * j, 16)] for j in range(_D // 16)]
            for c in range(_C):
                row = b * _C + c
                acc = w[0] * crows[row, pl.ds(0, 16)]
                for j in range(1, _D // 16):
                    acc = acc + w[j] * crows[row, pl.ds(16 * j, 16)]
                idx = jnp.full((16,), row, jnp.int32)
                plsc.addupdate_scatter(outbuf, [idx], acc)
            return carry

        lax.fori_loop(0, _CB, body, 0)
        pltpu.sync_copy(outbuf, out_hbm.at[pl.ds(b0 * _C, _PAIRS)])


@jax.jit
def kernel(target, context, target_table, context_table):
    tgt = target.astype(jnp.int32)
    ctx = context.reshape(-1).astype(jnp.int32)
    mesh = plsc.VectorSubcoreMesh(core_axis_name="c", subcore_axis_name="s",
                                  num_cores=_NC, num_subcores=_NS)
    out_flat = pl.kernel(
        _sc_body,
        out_type=jax.ShapeDtypeStruct((_B * _C,), jnp.float32),
        mesh=mesh,
        compiler_params=pltpu.CompilerParams(needs_layout_passes=False,
                                             use_tc_tiling_on_sc=True),
        scratch_types=[
            pltpu.VMEM((_CB,), jnp.int32),
            pltpu.VMEM((_PAIRS,), jnp.int32),
            pltpu.VMEM((_CB, _D), jnp.float32),
            pltpu.VMEM((_PAIRS, _D), jnp.float32),
            pltpu.VMEM((_PAIRS,), jnp.float32),
            pltpu.SemaphoreType.DMA,
        ],
    )(tgt, ctx, target_table, jnp.reshape(context_table, (125000, 8, _D)))
    return out_flat.reshape(_B, _C)


# speedup vs baseline: 1.9725x; 1.0304x over previous
import jax
import jax.numpy as jnp
from jax import lax
from jax.experimental import pallas as pl
from jax.experimental.pallas import tpu as pltpu
from jax.experimental.pallas import tpu_sc as plsc

_B = 16384
_C = 5
_D = 64
_NC = 2
_NS = 16
_NW = _NC * _NS
_BPW = _B // _NW
_CB = 128
_NCHUNK = _BPW // _CB
_PAIRS = _CB * _C


def _sc_body(tgt_hbm, ctx_hbm, ttab_hbm, ctab3_hbm, out_hbm,
             tidx, cidx, trows, crows, outbuf, sem):
    ctab_hbm = ctab3_hbm.reshape(1000000, _D)
    wid = lax.axis_index("s") * _NC + lax.axis_index("c")
    base = wid * _BPW
    for chunk in range(_NCHUNK):
        b0 = base + chunk * _CB
        pltpu.sync_copy(tgt_hbm.at[pl.ds(b0, _CB)], tidx)
        for j in range(_C):
            pltpu.sync_copy(ctx_hbm.at[pl.ds(j * _B + b0, _CB)],
                            cidx.at[pl.ds(j * _CB, _CB)])

        def fire_t(g, carry):
            v = tidx[pl.ds(g * 16, 16)]
            for i in range(16):
                pltpu.make_async_copy(ttab_hbm.at[pl.ds(v[i], 1)],
                                      trows.at[pl.ds(g * 16 + i, 1)],
                                      sem).start()
            return carry

        lax.fori_loop(0, _CB // 16, fire_t, 0)

        def fire_c(g, carry):
            v = cidx[pl.ds(g * 16, 16)]
            for i in range(16):
                pltpu.make_async_copy(ctab_hbm.at[pl.ds(v[i], 1)],
                                      crows.at[pl.ds(g * 16 + i, 1)],
                                      sem).start()
            return carry

        lax.fori_loop(0, _PAIRS // 16, fire_c, 0)

        def drain_t(k, carry):
            pltpu.make_async_copy(ttab_hbm.at[pl.ds(0, 1)],
                                  trows.at[pl.ds(k, 1)], sem).wait()
            return carry

        lax.fori_loop(0, _CB, drain_t, 0)

        def drain_c(k, carry):
            pltpu.make_async_copy(ctab_hbm.at[pl.ds(0, 1)],
                                  crows.at[pl.ds(k, 1)], sem).wait()
            return carry

        lax.fori_loop(0, _PAIRS, drain_c, 0)

        def zero(m, carry):
            outbuf[lax.shift_right_logical(m, 3),
                   pl.ds((m % 8) * 16, 16)] = jnp.zeros((16,), jnp.float32)
            return carry

        lax.fori_loop(0, _PAIRS // 16, zero, 0)

        def body(b, carry):
            w = [trows[b, pl.ds(16 * j, 16)] for j in range(_D // 16)]
            bf = jnp.full((16,), b, jnp.int32)
            for c in range(_C):
                row = c * _CB + b
                acc = w[0] * crows[row, pl.ds(0, 16)]
                for j in range(1, _D // 16):
                    acc = acc + w[j] * crows[row, pl.ds(16 * j, 16)]
                cf = jnp.full((16,), c, jnp.int32)
                plsc.addupdate_scatter(outbuf, [cf, bf], acc)
            return carry

        lax.fori_loop(0, _CB, body, 0)
        for j in range(_C):
            pltpu.sync_copy(outbuf.at[pl.ds(j, 1)],
                            out_hbm.at[pl.ds(j, 1), pl.ds(b0, _CB)])


@jax.jit
def kernel(target, context, target_table, context_table):
    tgt = target.astype(jnp.int32)
    ctx = context.T.reshape(-1).astype(jnp.int32)
    mesh = plsc.VectorSubcoreMesh(core_axis_name="c", subcore_axis_name="s",
                                  num_cores=_NC, num_subcores=_NS)
    out5 = pl.kernel(
        _sc_body,
        out_type=jax.ShapeDtypeStruct((_C, _B), jnp.float32),
        mesh=mesh,
        compiler_params=pltpu.CompilerParams(needs_layout_passes=False,
                                             use_tc_tiling_on_sc=True),
        scratch_types=[
            pltpu.VMEM((_CB,), jnp.int32),
            pltpu.VMEM((_PAIRS,), jnp.int32),
            pltpu.VMEM((_CB, _D), jnp.float32),
            pltpu.VMEM((_PAIRS, _D), jnp.float32),
            pltpu.VMEM((_C, _CB), jnp.float32),
            pltpu.SemaphoreType.DMA,
        ],
    )(tgt, ctx, target_table, jnp.reshape(context_table, (125000, 8, _D)))
    return out5.T
